# baseline (speedup 1.0000x reference)
import functools

import jax
import jax.numpy as jnp
import numpy as np
from jax import lax
from jax.experimental import pallas as pl
from jax.experimental.pallas import tpu as pltpu
from jax.experimental.pallas import tpu_sc as plsc

NUM_POSITIONS = 32768
EMB_DIM = 64
LANES = 16
IDX_CHUNK = 128


@functools.lru_cache(maxsize=None)
def _positions(seq_len: int) -> np.ndarray:
    with jax.ensure_compile_time_eval():
        pkey = jax.random.key(42)
        perm = np.asarray(jax.random.permutation(pkey, NUM_POSITIONS))
    return np.sort(perm[:seq_len]).astype(np.int32)


@functools.lru_cache(maxsize=None)
def _build_sc_gather(L: int, D: int):
    info = plsc.get_sparse_core_info()
    num_workers = info.num_cores * info.num_subcores
    RPW = L // num_workers
    chunks = RPW // IDX_CHUNK
    pos = _positions(L)
    pair_np = (pos // 2).reshape(-1, IDX_CHUNK)
    phase_np = (pos % 2).reshape(-1, IDX_CHUNK).astype(np.int32)
    mesh = plsc.VectorSubcoreMesh(core_axis_name="c", subcore_axis_name="s")

    @functools.partial(
        pl.kernel,
        out_type=jax.ShapeDtypeStruct((L * D,), jnp.float32),
        mesh=mesh,
        scratch_types=[
            pltpu.VMEM((chunks, IDX_CHUNK), jnp.int32),
            pltpu.VMEM((chunks, IDX_CHUNK), jnp.int32),
            pltpu.VMEM((RPW, 2 * D), jnp.float32),
            pltpu.VMEM((RPW * D,), jnp.float32),
            pltpu.SemaphoreType.DMA,
        ],
        compiler_params=pltpu.CompilerParams(needs_layout_passes=False),
    )
    def sc_gather(pair_hbm, phase_hbm, table_hbm, out_hbm, idx_v, ph_v,
                  pairs_v, rows_v, sem):
        wid = lax.axis_index("s") * info.num_cores + lax.axis_index("c")
        pltpu.sync_copy(pair_hbm.at[pl.ds(wid * chunks, chunks)], idx_v)
        gathers = [
            pltpu.async_copy(
                table_hbm.at[idx_v.at[j]],
                pairs_v.at[pl.ds(j * IDX_CHUNK, IDX_CHUNK)],
                sem,
            )
            for j in range(chunks)
        ]
        pltpu.sync_copy(phase_hbm.at[pl.ds(wid * chunks, chunks)], ph_v)
        for g in gathers:
            g.wait()
        iota = lax.iota(jnp.int32, LANES)
        gbases = [
            ph_v[j, pl.ds(b * LANES, LANES)] * D
            for j in range(chunks)
            for b in range(IDX_CHUNK // LANES)
        ]

        def body(col, c16, gbases=gbases):
            fl16 = iota * D + c16
            for blk, gbase in enumerate(gbases):
                o16 = iota + blk * LANES
                vals = plsc.load_gather(pairs_v, [o16, gbase + c16])
                plsc.store_scatter(rows_v, [fl16 + blk * (LANES * D)], vals)
            return (c16 + 1) & (D - 1)

        lax.fori_loop(0, D, body, iota, unroll=4)
        pltpu.async_copy(
            rows_v, out_hbm.at[pl.ds(wid * (RPW * D), RPW * D)], sem).wait()

    def run(table):
        table2 = table.reshape(NUM_POSITIONS // 2, 2 * D)
        return sc_gather(jnp.asarray(pair_np), jnp.asarray(phase_np), table2)

    return run


def kernel(x, table):
    B, L = x.shape
    D = table.shape[1]
    flat = _build_sc_gather(L, D)(table)
    return jnp.broadcast_to(flat.reshape(1, L, D), (B, L, D))


# speedup vs baseline: 1.1842x; 1.1842x over previous
import functools

import jax
import jax.numpy as jnp
import numpy as np
from jax import lax
from jax.experimental import pallas as pl
from jax.experimental.pallas import tpu as pltpu
from jax.experimental.pallas import tpu_sc as plsc

NUM_POSITIONS = 32768
EMB_DIM = 64
CHUNK = 64
LANES = 16


@functools.lru_cache(maxsize=None)
def _positions(seq_len: int) -> np.ndarray:
    with jax.ensure_compile_time_eval():
        pkey = jax.random.key(42)
        perm = np.asarray(jax.random.permutation(pkey, NUM_POSITIONS))
    return np.sort(perm[:seq_len]).astype(np.int32)


@functools.lru_cache(maxsize=None)
def _metadata(seq_len: int, num_workers: int):
    pos = _positions(seq_len)
    rows_per_worker = seq_len // num_workers
    nch = rows_per_worker // CHUNK
    chunks = pos.reshape(num_workers, nch, CHUNK)
    lo = (chunks[:, :, 0] // 8) * 8
    span = int(np.max(chunks[:, :, -1] - lo + 1))
    span = ((span + 7) // 8) * 8
    lo = np.minimum(lo, NUM_POSITIONS - span)
    rowoff = (chunks - lo[:, :, None]).astype(np.int32)
    return (lo.astype(np.int64), rowoff.reshape(num_workers, rows_per_worker),
            span, nch, rows_per_worker)


@functools.lru_cache(maxsize=None)
def _build_sc_gather(L: int, D: int):
    info = plsc.get_sparse_core_info()
    num_workers = info.num_cores * info.num_subcores
    lo_np, rowoff_np, SPAN, NCH, RPW = _metadata(L, num_workers)
    mesh = plsc.VectorSubcoreMesh(core_axis_name="c", subcore_axis_name="s")

    @functools.partial(
        pl.kernel,
        out_type=jax.ShapeDtypeStruct((L, D), jnp.float32),
        mesh=mesh,
        scratch_types=[
            pltpu.VMEM((RPW,), jnp.int32),
            pltpu.VMEM((SPAN, D), jnp.float32),
            pltpu.VMEM((SPAN, D), jnp.float32),
            pltpu.VMEM((CHUNK, D), jnp.float32),
            pltpu.VMEM((CHUNK, D), jnp.float32),
            pltpu.SemaphoreType.DMA,
            pltpu.SemaphoreType.DMA,
            pltpu.SemaphoreType.DMA,
            pltpu.SemaphoreType.DMA,
        ],
        compiler_params=pltpu.CompilerParams(needs_layout_passes=False),
    )
    def sc_gather(rowoff_hbm, table_hbm, out_hbm, ro_v, buf0, buf1, rows0,
                  rows1, semw0, semw1, semr0, semr1):
        wid = lax.axis_index("s") * info.num_cores + lax.axis_index("c")
        base = wid * RPW
        bufs, semws = (buf0, buf1), (semw0, semw1)
        rows, semrs = (rows0, rows1), (semr0, semr1)

        def win_dma(c, buf, sem):
            for k in range(num_workers):
                @pl.when(wid == k)
                def _(k=k, c=c):
                    pltpu.async_copy(
                        table_hbm.at[pl.ds(int(lo_np[k, c]), SPAN)], buf, sem)

        win_dma(0, buf0, semw0)
        win_dma(1, buf1, semw1)
        pltpu.sync_copy(rowoff_hbm.at[wid], ro_v)
        for c in range(NCH):
            pb = c % 2
            buf, semw = bufs[pb], semws[pb]
            rbuf, semr = rows[pb], semrs[pb]
            pltpu.make_async_copy(
                table_hbm.at[pl.ds(0, SPAN)], buf, semw).wait()
            if c >= 2:
                pltpu.make_async_copy(
                    rbuf, out_hbm.at[pl.ds(0, CHUNK)], semr).wait()
            blocks = [
                (ro_v[pl.ds(c * CHUNK + b * LANES, LANES)],
                 lax.iota(jnp.int32, LANES) + b * LANES)
                for b in range(CHUNK // LANES)
            ]

            def body(col, _, blocks=blocks, buf=buf, rbuf=rbuf):
                c16 = (lax.iota(jnp.int32, LANES) + col) & (D - 1)
                for r16, o16 in blocks:
                    vals = plsc.load_gather(buf, [r16, c16])
                    plsc.store_scatter(rbuf, [o16, c16], vals)
                return 0

            lax.fori_loop(0, D, body, 0, unroll=4)
            if c + 2 < NCH:
                win_dma(c + 2, buf, semw)
            pltpu.async_copy(
                rbuf, out_hbm.at[pl.ds(base + c * CHUNK, CHUNK)], semr)
        for pb in range(2):
            pltpu.make_async_copy(
                rows[pb], out_hbm.at[pl.ds(0, CHUNK)], semrs[pb]).wait()

    def run(table):
        return sc_gather(jnp.asarray(rowoff_np), table)

    return run


def kernel(x, table):
    B, L = x.shape
    D = table.shape[1]
    rows = _build_sc_gather(L, D)(table)
    return jnp.broadcast_to(rows[None], (B, L, D))
